# baseline (speedup 1.0000x reference)
import jax
import jax.numpy as jnp
from jax import lax
from jax.experimental import pallas as pl
from jax.experimental.pallas import tpu as pltpu
from jax.experimental.pallas import tpu_sc as plsc

B = 1024
L = 50
P = L * L
H = 8
V = 1025
LANES = 16

NC = 2
NS = 16
NW = NC * NS
CP = 4
NCHUNK = P // CP
CPW = -(-NCHUNK // NW)
CHUNK_IDX = CP * B
CHUNK_OUT = CP * H * B
VPB = B // LANES


def _ptable_body(emb_ref, w_ref, b_ref, out_ref):
    out_ref[...] = (
        jax.lax.dot_general(
            w_ref[...], emb_ref[...], (((0,), (1,)), ((), ())),
            preferred_element_type=jnp.float32,
        )
        + b_ref[...]
    )


def _make_ptable(emb_table, w, bias):
    return pl.pallas_call(
        _ptable_body,
        out_shape=jax.ShapeDtypeStruct((H, V), jnp.float32),
    )(emb_table, w, bias.reshape(H, 1))


def _sc_body(ptable_hbm, idx_hbm, out_hbm,
             table_v, idx_v0, idx_v1, out_v0, out_v1,
             sem_i0, sem_i1, sem_o0, sem_o1):
    wid = lax.axis_index("s") * NC + lax.axis_index("c")
    idx_bufs = (idx_v0, idx_v1)
    out_bufs = (out_v0, out_v1)
    isems = (sem_i0, sem_i1)
    osems = (sem_o0, sem_o1)

    pltpu.sync_copy(ptable_hbm, table_v)

    g0 = wid * CPW

    def idx_src(g):
        return idx_hbm.at[pl.ds(g * CHUNK_IDX, CHUNK_IDX)]

    def out_dst(g):
        return out_hbm.at[pl.ds(g * CHUNK_OUT, CHUNK_OUT)]

    def start_idx(g, buf):
        pltpu.make_async_copy(idx_src(g), idx_bufs[buf], isems[buf]).start()

    def wait_idx(g, buf):
        pltpu.make_async_copy(idx_src(g), idx_bufs[buf], isems[buf]).wait()

    def start_out(g, buf):
        pltpu.make_async_copy(out_bufs[buf], out_dst(g), osems[buf]).start()

    def wait_out(g, buf):
        pltpu.make_async_copy(out_bufs[buf], out_dst(g), osems[buf]).wait()

    def compute(idx_ref, out_ref):
        @plsc.parallel_loop(0, CP * VPB, unroll=8)
        def _(v):
            idxv = idx_ref[pl.ds(v * LANES, LANES)]
            base = (v // 8) * (H * 128) + (v % 8) * LANES
            for h in range(H):
                vals = plsc.load_gather(table_v, [idxv + h * V])
                out_ref[pl.ds(base + h * 128, LANES)] = vals

    def guarded(g, fn, *args):
        @pl.when(g < NCHUNK)
        def _():
            fn(*args)

    guarded(g0, start_idx, g0, 0)
    guarded(g0 + 1, start_idx, g0 + 1, 1)
    for c in range(CPW):
        g = g0 + c
        buf = c % 2

        if c >= 2:
            guarded(g - 2, wait_out, g - 2, buf)

        @pl.when(g < NCHUNK)
        def _(g=g, buf=buf, c=c):
            wait_idx(g, buf)
            compute(idx_bufs[buf], out_bufs[buf])
            start_out(g, buf)
            if c + 2 < CPW:
                guarded(g + 2, start_idx, g + 2, buf)

    guarded(g0 + CPW - 2, wait_out, g0 + CPW - 2, (CPW - 2) % 2)
    guarded(g0 + CPW - 1, wait_out, g0 + CPW - 1, (CPW - 1) % 2)


_sc_gather = pl.kernel(
    _sc_body,
    out_type=jax.ShapeDtypeStruct((P * H * B,), jnp.float32),
    mesh=plsc.VectorSubcoreMesh(core_axis_name="c", subcore_axis_name="s"),
    compiler_params=pltpu.CompilerParams(needs_layout_passes=False),
    scratch_types=[
        pltpu.VMEM((H * V,), jnp.float32),
        pltpu.VMEM((CHUNK_IDX,), jnp.int32),
        pltpu.VMEM((CHUNK_IDX,), jnp.int32),
        pltpu.VMEM((CHUNK_OUT,), jnp.float32),
        pltpu.VMEM((CHUNK_OUT,), jnp.float32),
        pltpu.SemaphoreType.DMA,
        pltpu.SemaphoreType.DMA,
        pltpu.SemaphoreType.DMA,
        pltpu.SemaphoreType.DMA,
    ],
)


def kernel(input_time_matrix, emb_table, W, b):
    idx_t = jnp.transpose(input_time_matrix, (1, 2, 0)).reshape(-1)
    if idx_t.dtype != jnp.int32:
        idx_t = idx_t.astype(jnp.int32)
    ptable = _make_ptable(emb_table, W, b).reshape(-1)
    out_flat = _sc_gather(ptable, idx_t)
    out = out_flat.reshape(L, L, B // 128, H, 128)
    return jnp.transpose(out, (2, 4, 3, 0, 1)).reshape(B, H, L, L)


# speedup vs baseline: 1.0454x; 1.0454x over previous
import jax
import jax.numpy as jnp
from jax import lax
from jax.experimental import pallas as pl
from jax.experimental.pallas import tpu as pltpu
from jax.experimental.pallas import tpu_sc as plsc

B = 1024
L = 50
P = L * L
H = 8
V = 1025
LANES = 16

NC = 2
NS = 16
NW = NC * NS
CP = 4
NCHUNK = P // CP
CPW = -(-NCHUNK // NW)
CHUNK_IDX = CP * B
CHUNK_OUT = CP * H * B
VPB = B // LANES


def _ptable_body(embt_ref, wt_ref, b_ref, out_ref):
    out_ref[...] = (
        jax.lax.dot_general(
            wt_ref[...], embt_ref[...], (((1,), (0,)), ((), ())),
            preferred_element_type=jnp.float32,
        )
        + b_ref[...]
    )


def _make_ptable(emb_table, w, bias):
    return pl.pallas_call(
        _ptable_body,
        out_shape=jax.ShapeDtypeStruct((H, V), jnp.float32),
    )(emb_table.T, w.T, bias.reshape(H, 1))


def _sc_body(ptable_hbm, idx_hbm, out_hbm,
             table_v, idx_v0, idx_v1, out_v0, out_v1,
             sem_i0, sem_i1, sem_o0, sem_o1):
    wid = lax.axis_index("s") * NC + lax.axis_index("c")
    idx_bufs = (idx_v0, idx_v1)
    out_bufs = (out_v0, out_v1)
    isems = (sem_i0, sem_i1)
    osems = (sem_o0, sem_o1)

    pltpu.sync_copy(ptable_hbm, table_v)

    g0 = wid * CPW

    def idx_src(g):
        return idx_hbm.at[pl.ds(g * CHUNK_IDX, CHUNK_IDX)]

    def out_dst(g):
        return out_hbm.at[pl.ds(g * CHUNK_OUT, CHUNK_OUT)]

    def start_idx(g, buf):
        pltpu.make_async_copy(idx_src(g), idx_bufs[buf], isems[buf]).start()

    def wait_idx(g, buf):
        pltpu.make_async_copy(idx_src(g), idx_bufs[buf], isems[buf]).wait()

    def start_out(g, buf):
        pltpu.make_async_copy(out_bufs[buf], out_dst(g), osems[buf]).start()

    def wait_out(g, buf):
        pltpu.make_async_copy(out_bufs[buf], out_dst(g), osems[buf]).wait()

    def compute(idx_ref, out_ref):
        @plsc.parallel_loop(0, CP * VPB, unroll=8)
        def _(v):
            idxv = idx_ref[pl.ds(v * LANES, LANES)]
            base = (v // 8) * (H * 128) + (v % 8) * LANES
            for h in range(H):
                vals = plsc.load_gather(table_v, [idxv + h * V])
                out_ref[pl.ds(base + h * 128, LANES)] = vals

    def guarded(g, fn, *args):
        @pl.when(g < NCHUNK)
        def _():
            fn(*args)

    guarded(g0, start_idx, g0, 0)
    guarded(g0 + 1, start_idx, g0 + 1, 1)
    for c in range(CPW):
        g = g0 + c
        buf = c % 2

        if c >= 2:
            guarded(g - 2, wait_out, g - 2, buf)

        @pl.when(g < NCHUNK)
        def _(g=g, buf=buf, c=c):
            wait_idx(g, buf)
            compute(idx_bufs[buf], out_bufs[buf])
            start_out(g, buf)
            if c + 2 < CPW:
                guarded(g + 2, start_idx, g + 2, buf)

    guarded(g0 + CPW - 2, wait_out, g0 + CPW - 2, (CPW - 2) % 2)
    guarded(g0 + CPW - 1, wait_out, g0 + CPW - 1, (CPW - 1) % 2)


_sc_gather = pl.kernel(
    _sc_body,
    out_type=jax.ShapeDtypeStruct((P * H * B,), jnp.float32),
    mesh=plsc.VectorSubcoreMesh(core_axis_name="c", subcore_axis_name="s"),
    compiler_params=pltpu.CompilerParams(needs_layout_passes=False),
    scratch_types=[
        pltpu.VMEM((H * V,), jnp.float32),
        pltpu.VMEM((CHUNK_IDX,), jnp.int32),
        pltpu.VMEM((CHUNK_IDX,), jnp.int32),
        pltpu.VMEM((CHUNK_OUT,), jnp.float32),
        pltpu.VMEM((CHUNK_OUT,), jnp.float32),
        pltpu.SemaphoreType.DMA,
        pltpu.SemaphoreType.DMA,
        pltpu.SemaphoreType.DMA,
        pltpu.SemaphoreType.DMA,
    ],
)


def kernel(input_time_matrix, emb_table, W, b):
    idx_t = jnp.transpose(input_time_matrix, (1, 2, 0)).reshape(-1)
    if idx_t.dtype != jnp.int32:
        idx_t = idx_t.astype(jnp.int32)
    ptable = _make_ptable(emb_table, W, b).reshape(-1)
    out_flat = _sc_gather(ptable, idx_t)
    out = out_flat.reshape(L, L, B // 128, H, 128)
    return jnp.transpose(out, (2, 4, 3, 0, 1)).reshape(B, H, L, L)


# speedup vs baseline: 1.0532x; 1.0075x over previous
import jax
import jax.numpy as jnp
from jax import lax
from jax.experimental import pallas as pl
from jax.experimental.pallas import tpu as pltpu
from jax.experimental.pallas import tpu_sc as plsc

B = 1024
L = 50
P = L * L
H = 8
V = 1025
LANES = 16

NC = 2
NS = 16
NW = NC * NS
CP = 5
NCHUNK = P // CP
CPW = -(-NCHUNK // NW)
CHUNK_IDX = CP * B
CHUNK_OUT = CP * H * B
VPB = B // LANES


def _ptable_body(embt_ref, wt_ref, b_ref, out_ref):
    out_ref[...] = (
        jax.lax.dot_general(
            wt_ref[...], embt_ref[...], (((1,), (0,)), ((), ())),
            preferred_element_type=jnp.float32,
        )
        + b_ref[...]
    )


def _make_ptable(emb_table, w, bias):
    return pl.pallas_call(
        _ptable_body,
        out_shape=jax.ShapeDtypeStruct((H, V), jnp.float32),
    )(emb_table.T, w.T, bias.reshape(H, 1))


def _sc_body(ptable_hbm, idx_hbm, out_hbm,
             table_v, idx_v0, idx_v1, out_v0, out_v1,
             sem_i0, sem_i1, sem_o0, sem_o1):
    wid = lax.axis_index("s") * NC + lax.axis_index("c")
    idx_bufs = (idx_v0, idx_v1)
    out_bufs = (out_v0, out_v1)
    isems = (sem_i0, sem_i1)
    osems = (sem_o0, sem_o1)

    pltpu.sync_copy(ptable_hbm, table_v)

    g0 = wid * CPW

    def idx_src(g):
        return idx_hbm.at[pl.ds(g * CHUNK_IDX, CHUNK_IDX)]

    def out_dst(g):
        return out_hbm.at[pl.ds(g * CHUNK_OUT, CHUNK_OUT)]

    def start_idx(g, buf):
        pltpu.make_async_copy(idx_src(g), idx_bufs[buf], isems[buf]).start()

    def wait_idx(g, buf):
        pltpu.make_async_copy(idx_src(g), idx_bufs[buf], isems[buf]).wait()

    def start_out(g, buf):
        pltpu.make_async_copy(out_bufs[buf], out_dst(g), osems[buf]).start()

    def wait_out(g, buf):
        pltpu.make_async_copy(out_bufs[buf], out_dst(g), osems[buf]).wait()

    def compute(idx_ref, out_ref):
        @plsc.parallel_loop(0, CP * VPB, unroll=8)
        def _(v):
            idxv = idx_ref[pl.ds(v * LANES, LANES)]
            base = (v // 8) * (H * 128) + (v % 8) * LANES
            for h in range(H):
                vals = plsc.load_gather(table_v, [idxv + h * V])
                out_ref[pl.ds(base + h * 128, LANES)] = vals

    def guarded(g, fn, *args):
        @pl.when(g < NCHUNK)
        def _():
            fn(*args)

    guarded(g0, start_idx, g0, 0)
    guarded(g0 + 1, start_idx, g0 + 1, 1)
    for c in range(CPW):
        g = g0 + c
        buf = c % 2

        if c >= 2:
            guarded(g - 2, wait_out, g - 2, buf)

        @pl.when(g < NCHUNK)
        def _(g=g, buf=buf, c=c):
            wait_idx(g, buf)
            compute(idx_bufs[buf], out_bufs[buf])
            start_out(g, buf)
            if c + 2 < CPW:
                guarded(g + 2, start_idx, g + 2, buf)

    guarded(g0 + CPW - 2, wait_out, g0 + CPW - 2, (CPW - 2) % 2)
    guarded(g0 + CPW - 1, wait_out, g0 + CPW - 1, (CPW - 1) % 2)


_sc_gather = pl.kernel(
    _sc_body,
    out_type=jax.ShapeDtypeStruct((P * H * B,), jnp.float32),
    mesh=plsc.VectorSubcoreMesh(core_axis_name="c", subcore_axis_name="s"),
    compiler_params=pltpu.CompilerParams(needs_layout_passes=False),
    scratch_types=[
        pltpu.VMEM((H * V,), jnp.float32),
        pltpu.VMEM((CHUNK_IDX,), jnp.int32),
        pltpu.VMEM((CHUNK_IDX,), jnp.int32),
        pltpu.VMEM((CHUNK_OUT,), jnp.float32),
        pltpu.VMEM((CHUNK_OUT,), jnp.float32),
        pltpu.SemaphoreType.DMA,
        pltpu.SemaphoreType.DMA,
        pltpu.SemaphoreType.DMA,
        pltpu.SemaphoreType.DMA,
    ],
)


def kernel(input_time_matrix, emb_table, W, b):
    idx_t = jnp.transpose(input_time_matrix, (1, 2, 0)).reshape(-1)
    if idx_t.dtype != jnp.int32:
        idx_t = idx_t.astype(jnp.int32)
    ptable = _make_ptable(emb_table, W, b).reshape(-1)
    out_flat = _sc_gather(ptable, idx_t)
    out = out_flat.reshape(L, L, B // 128, H, 128)
    return jnp.transpose(out, (2, 4, 3, 0, 1)).reshape(B, H, L, L)
